# baseline (speedup 1.0000x reference)
import functools

import jax
import jax.numpy as jnp
from jax import lax
from jax.experimental import pallas as pl
from jax.experimental.pallas import tpu as pltpu
from jax.experimental.pallas import tpu_sc as plsc

_NUM_CORES = 2
_NUM_SUBCORES = 16
_NUM_WORKERS = _NUM_CORES * _NUM_SUBCORES
_CHUNK = 640
_NBUF = 4


@functools.lru_cache(maxsize=None)
def _make_gather(n, d, chunk):
    per_w = n // _NUM_WORKERS
    n_chunks = per_w // chunk
    assert n_chunks % _NBUF == 0 and n_chunks >= 2 * _NBUF
    mesh = plsc.VectorSubcoreMesh(
        core_axis_name="c",
        subcore_axis_name="s",
        num_cores=_NUM_CORES,
        num_subcores=_NUM_SUBCORES,
    )

    @functools.partial(
        pl.kernel,
        mesh=mesh,
        compiler_params=pltpu.CompilerParams(use_tc_tiling_on_sc=False),
        out_type=jax.ShapeDtypeStruct((n, d), jnp.float32),
        scratch_types=[
            pltpu.VMEM((per_w,), jnp.int32),
            pltpu.VMEM((_NBUF, chunk, d), jnp.float32),
            [pltpu.SemaphoreType.DMA] * _NBUF,
            [pltpu.SemaphoreType.DMA] * _NBUF,
        ],
    )
    def gather_kernel(idx_hbm, table_hbm, out_hbm, idx_v, rows_v, gsem, wsem):
        wid = lax.axis_index("s") * _NUM_CORES + lax.axis_index("c")
        base = wid * per_w

        pltpu.sync_copy(idx_hbm.at[pl.ds(pl.multiple_of(base, 8), per_w)], idx_v)

        def issue_gather(g, b):
            pltpu.async_copy(
                table_hbm.at[idx_v.at[pl.ds(g * chunk, chunk)]],
                rows_v.at[b], gsem[b])

        def wait_gather(b):
            pltpu.make_async_copy(
                table_hbm.at[idx_v.at[pl.ds(0, chunk)]], rows_v.at[b], gsem[b]
            ).wait()

        def issue_write(g, b):
            off = pl.multiple_of(base + g * chunk, 8)
            pltpu.async_copy(rows_v.at[b], out_hbm.at[pl.ds(off, chunk)], wsem[b])

        def wait_write(b):
            pltpu.make_async_copy(
                rows_v.at[b], out_hbm.at[pl.ds(0, chunk)], wsem[b]).wait()

        issue_gather(0, 0)
        issue_gather(1, 1)

        def body(i, carry):
            for b in range(_NBUF):
                g = _NBUF * i + b
                pre = g + 2
                wait_gather(b)
                issue_write(g, b)
                @pl.when(pre < n_chunks)
                def _():
                    @pl.when(g >= 2)
                    def _():
                        wait_write((b + 2) % _NBUF)
                    issue_gather(pre, (b + 2) % _NBUF)
            return carry

        lax.fori_loop(0, n_chunks // _NBUF, body, 0)
        for b in range(_NBUF):
            wait_write(b)

    return gather_kernel


def kernel(X, weight):
    n = X.size
    flat_idx = X.reshape(n)
    out = _make_gather(n, weight.shape[1], _CHUNK)(flat_idx, weight)
    return out.reshape(X.shape + (weight.shape[1],))
